# baseline (speedup 1.0000x reference)
import functools

import jax
import jax.numpy as jnp
from jax import lax
from jax.experimental import pallas as pl
from jax.experimental.pallas import tpu as pltpu
from jax.experimental.pallas import tpu_sc as plsc

B = 64
S = 1024
D = 768
N = B * S
L = 16

NUM_WORKERS = 32
PER_W = N // NUM_WORKERS
REPS = PER_W // S
C = 32
NPCHUNKS = S // C
NSTEPS = NPCHUNKS * REPS


def _emb_body(x_hbm, tok_hbm, pos_hbm, out_hbm,
              idx_v, rows_v, pos_v, gsem, osem, psem):
    wid = lax.axis_index("s") * 2 + lax.axis_index("c")
    base = wid * PER_W

    pltpu.sync_copy(x_hbm.at[pl.ds(base, PER_W)], idx_v)
    pltpu.sync_copy(pos_hbm.at[pl.ds(0, C)], pos_v.at[0])
    pltpu.async_copy(tok_hbm.at[idx_v.at[pl.ds(0, C)]], rows_v.at[0], gsem)

    def step(s, carry):
        b = s & 1
        nb = 1 - b
        c = s >> 1
        pb = c & 1
        p0 = c * C
        start = base + b * S + p0

        @pl.when(s + 1 < NSTEPS)
        def _():
            @pl.when(s >= 1)
            def _():
                pltpu.make_async_copy(
                    rows_v.at[nb], out_hbm.at[pl.ds(0, C)], osem).wait()
            c1 = (s + 1) >> 1
            off1 = nb * S + c1 * C
            pltpu.async_copy(
                tok_hbm.at[idx_v.at[pl.ds(off1, C)]], rows_v.at[nb], gsem)

        @pl.when((b == 0) & (c + 1 < NPCHUNKS))
        def _():
            pltpu.async_copy(
                pos_hbm.at[pl.ds(p0 + C, C)], pos_v.at[1 - pb], psem)

        pltpu.make_async_copy(
            tok_hbm.at[idx_v.at[pl.ds(0, C)]], rows_v.at[b], gsem).wait()

        @pl.when((b == 0) & (s > 0))
        def _():
            pltpu.make_async_copy(
                pos_hbm.at[pl.ds(0, C)], pos_v.at[pb], psem).wait()

        def addrow(j, carry2):
            for k in range(D // L):
                sl = pl.ds(k * L, L)
                plsc.addupdate(rows_v.at[b, j, sl], pos_v[pb, j, sl])
            return carry2

        lax.fori_loop(0, C, addrow, 0)

        pltpu.async_copy(rows_v.at[b], out_hbm.at[pl.ds(start, C)], osem)
        return carry

    lax.fori_loop(0, NSTEPS, step, 0)

    pltpu.make_async_copy(rows_v.at[0], out_hbm.at[pl.ds(0, C)], osem).wait()
    pltpu.make_async_copy(rows_v.at[1], out_hbm.at[pl.ds(0, C)], osem).wait()


@jax.jit
def _emb(x_flat, token_emb, pos2d):
    mesh = plsc.VectorSubcoreMesh(core_axis_name="c", subcore_axis_name="s")
    f = functools.partial(
        pl.kernel,
        out_type=jax.ShapeDtypeStruct((N, D), jnp.float32),
        mesh=mesh,
        scratch_types=[
            pltpu.VMEM((PER_W,), jnp.int32),
            pltpu.VMEM((2, C, D), jnp.float32),
            pltpu.VMEM((2, C, D), jnp.float32),
            pltpu.SemaphoreType.DMA,
            pltpu.SemaphoreType.DMA,
            pltpu.SemaphoreType.DMA,
        ],
    )(_emb_body)
    return f(x_flat, token_emb, pos2d)


def kernel(x, token_emb, pos_emb):
    x_flat = x.reshape(N)
    pos2d = pos_emb.reshape(S, D)
    out = _emb(x_flat, token_emb, pos2d)
    return out.reshape(B, S, D)
